# Initial kernel scaffold; baseline (speedup 1.0000x reference)
import jax
import jax.numpy as jnp
from jax.experimental import pallas as pl


def kernel(x, edge_index, W_l, b_l, W_r, b_r, att, bias, gn_weight, gn_bias, gn_mean_scale):
    raise NotImplementedError("write your pallas kernel here")


# speedup vs baseline: 11.1498x; 11.1498x over previous
import functools

import jax
import jax.numpy as jnp
from jax import lax
from jax.experimental import pallas as pl
from jax.experimental.pallas import tpu as pltpu
from jax.experimental.pallas import tpu_sc as plsc

N = 10000
E = 320000
C = 128
H = 4
HC = H * C

NC = 2
NS = 16
NW = NC * NS
EPT = 10016
B = 32
NBLK = EPT // B
E_PAD = NW * EPT
ND = H * EPT
NP_OUT = 10240

_mesh = plsc.VectorSubcoreMesh(
    core_axis_name="c", subcore_axis_name="s", num_cores=NC, num_subcores=NS)


def _mm_body(x_ref, wl_ref, wr_ref, bl_ref, br_ref, xl_ref, xr_ref):
    xv = x_ref[...]
    xl_ref[...] = jnp.dot(xv, wl_ref[...],
                          preferred_element_type=jnp.float32) + bl_ref[...]
    xr_ref[...] = jnp.dot(xv, wr_ref[...],
                          preferred_element_type=jnp.float32) + br_ref[...]


def _matmuls(x, W_l, W_r, b_l, b_r):
    blk = 1000
    grid = (N // blk,)
    return pl.pallas_call(
        _mm_body,
        grid=grid,
        in_specs=[
            pl.BlockSpec((blk, C), lambda i: (i, 0)),
            pl.BlockSpec((C, HC), lambda i: (0, 0)),
            pl.BlockSpec((C, HC), lambda i: (0, 0)),
            pl.BlockSpec((1, HC), lambda i: (0, 0)),
            pl.BlockSpec((1, HC), lambda i: (0, 0)),
        ],
        out_specs=[
            pl.BlockSpec((blk, HC), lambda i: (i, 0)),
            pl.BlockSpec((blk, HC), lambda i: (i, 0)),
        ],
        out_shape=[
            jax.ShapeDtypeStruct((N, HC), jnp.float32),
            jax.ShapeDtypeStruct((N, HC), jnp.float32),
        ],
    )(x, W_l, W_r, b_l.reshape(1, HC), b_r.reshape(1, HC))


def _p1_body(xl_hbm, xr_hbm, src_hbm, dst_hbm, att_hbm,
             ex_hbm, den_hbm,
             sidx, didx, xlbuf, xrbuf, lscr, exblk, att_v, den_v,
             sem1, sem2):
    c = lax.axis_index("c")
    s = lax.axis_index("s")
    wid = s * NC + c

    pltpu.sync_copy(att_hbm, att_v)
    attv = [att_v[pl.ds(16 * j, 16)] for j in range(32)]
    zeros16 = jnp.zeros((16,), jnp.float32)

    @pl.loop(0, ND // 16)
    def _zero(i):
        den_v[pl.ds(i * 16, 16)] = zeros16

    base0 = wid * EPT
    iota = lax.iota(jnp.int32, 16)

    @pl.loop(0, NBLK)
    def _blk(blk):
        base = base0 + blk * B
        pltpu.sync_copy(src_hbm.at[pl.ds(base, B)], sidx)
        pltpu.sync_copy(dst_hbm.at[pl.ds(base, B)], didx)
        cp1 = pltpu.async_copy(xl_hbm.at[sidx], xlbuf, sem1)
        cp2 = pltpu.async_copy(xr_hbm.at[didx], xrbuf, sem2)
        cp1.wait()
        cp2.wait()

        @pl.loop(0, B)
        def _edge(e):
            for h in range(H):
                acc = zeros16
                for j8 in range(8):
                    j = h * 8 + j8
                    sv = xlbuf[e, pl.ds(16 * j, 16)] + xrbuf[e, pl.ds(16 * j, 16)]
                    lv = jnp.maximum(sv, 0.2 * sv)
                    acc = acc + lv * attv[j]
                lscr[pl.ds(e * 64 + h * 16, 16)] = acc

        onehot0 = (iota == 0).astype(jnp.float32)
        for g in range(2):
            dstv = didx[pl.ds(g * 16, 16)]
            for h in range(H):
                tot = zeros16
                for j in range(16):
                    tot = tot + plsc.load_gather(
                        lscr, [iota * 64 + (g * 1024 + h * 16 + j)])
                exv = jnp.exp(tot)
                plsc.store_scatter(exblk, [iota * 4 + (g * 64 + h)], exv)
                for j in range(16):
                    idx = dstv[j] * 4 + h
                    plsc.addupdate(den_v.at[pl.ds(idx, 16)], exv[j] * onehot0)

        pltpu.sync_copy(exblk, ex_hbm.at[pl.ds(base * 4, B * 4)])

    pltpu.sync_copy(den_v, den_hbm.at[wid])


@functools.partial(
    pl.kernel,
    out_type=(
        jax.ShapeDtypeStruct((E_PAD * 4,), jnp.float32),
        jax.ShapeDtypeStruct((NW, ND), jnp.float32),
    ),
    mesh=_mesh,
    scratch_types=[
        pltpu.VMEM((B,), jnp.int32),
        pltpu.VMEM((B,), jnp.int32),
        pltpu.VMEM((B, HC), jnp.float32),
        pltpu.VMEM((B, HC), jnp.float32),
        pltpu.VMEM((B * 64,), jnp.float32),
        pltpu.VMEM((B * 4,), jnp.float32),
        pltpu.VMEM((HC,), jnp.float32),
        pltpu.VMEM((ND,), jnp.float32),
        pltpu.SemaphoreType.DMA,
        pltpu.SemaphoreType.DMA,
    ],
    compiler_params=pltpu.CompilerParams(needs_layout_passes=False),
)
def _phase1(*refs):
    _p1_body(*refs)


def _mid_body(den_ref, rden_ref):
    d = jnp.sum(den_ref[...], axis=0) + 1e-16
    rden_ref[...] = 1.0 / d


def _mid(den):
    return pl.pallas_call(
        _mid_body,
        out_shape=jax.ShapeDtypeStruct((ND // 128, 128), jnp.float32),
    )(den.reshape(NW, ND // 128, 128))


def _p2_body(ex_hbm, dst_hbm, rden_hbm, al_hbm, didx, exblk, rden_v):
    c = lax.axis_index("c")
    s = lax.axis_index("s")
    wid = s * NC + c
    base = wid * EPT

    pltpu.sync_copy(rden_hbm, rden_v)
    pltpu.sync_copy(dst_hbm.at[pl.ds(base, EPT)], didx)
    pltpu.sync_copy(ex_hbm.at[pl.ds(base * 4, EPT * 4)], exblk)
    iota = lax.iota(jnp.int32, 16)

    @pl.loop(0, EPT // 16)
    def _grp(g):
        dstv = didx[pl.ds(g * 16, 16)]
        dst4 = dstv * 4
        for h in range(H):
            exv = plsc.load_gather(exblk, [iota * 4 + (g * 64 + h)])
            rd = plsc.load_gather(rden_v, [dst4 + h])
            plsc.store_scatter(exblk, [iota * 4 + (g * 64 + h)], exv * rd)

    pltpu.sync_copy(exblk, al_hbm.at[pl.ds(base * 4, EPT * 4)])


@functools.partial(
    pl.kernel,
    out_type=jax.ShapeDtypeStruct((E_PAD * 4,), jnp.float32),
    mesh=_mesh,
    scratch_types=[
        pltpu.VMEM((EPT,), jnp.int32),
        pltpu.VMEM((EPT * 4,), jnp.float32),
        pltpu.VMEM((ND,), jnp.float32),
    ],
    compiler_params=pltpu.CompilerParams(needs_layout_passes=False),
)
def _phase2(*refs):
    _p2_body(*refs)


def _p3_body(xl_hbm, src_hbm, dst_hbm, al_hbm,
             outp_hbm,
             sidx, didx, xlbuf, ascr, vbuf, out_sh, sem1):
    c = lax.axis_index("c")
    s = lax.axis_index("s")
    wid = s * NC + c

    zeros16 = jnp.zeros((16,), jnp.float32)

    @pl.loop(0, B)
    def _zv(e):
        for j8 in range(8):
            vbuf[e, pl.ds(16 * j8, 16)] = zeros16

    @pl.loop(0, NP_OUT // (NS * B))
    def _zo(t):
        pltpu.sync_copy(vbuf, out_sh.at[pl.ds(s * (NP_OUT // NS) + t * B, B)])

    plsc.subcore_barrier()

    base0 = wid * EPT
    iota = lax.iota(jnp.int32, 16)

    @pl.loop(0, NBLK)
    def _blk(blk):
        base = base0 + blk * B
        pltpu.sync_copy(src_hbm.at[pl.ds(base, B)], sidx)
        pltpu.sync_copy(dst_hbm.at[pl.ds(base, B)], didx)
        pltpu.sync_copy(al_hbm.at[pl.ds(base * 4, B * 4)], ascr.at[pl.ds(0, B * 4)])
        pltpu.async_copy(xl_hbm.at[sidx], xlbuf, sem1).wait()

        @pl.loop(0, B)
        def _edge(e):
            av = ascr[pl.ds(e * 4, 16)]
            a0 = av[0]
            a1 = av[1]
            a2 = av[2]
            a3 = av[3]
            for c8 in range(8):
                v = (a0 * xlbuf[e, pl.ds(c8 * 16, 16)]
                     + a1 * xlbuf[e, pl.ds(128 + c8 * 16, 16)]
                     + a2 * xlbuf[e, pl.ds(256 + c8 * 16, 16)]
                     + a3 * xlbuf[e, pl.ds(384 + c8 * 16, 16)])
                vbuf[e, pl.ds(c8 * 16, 16)] = v

        pltpu.sync_copy(vbuf, out_sh.at[didx], add=True)

    plsc.subcore_barrier()
    rows = NP_OUT // NS
    pltpu.sync_copy(out_sh.at[pl.ds(s * rows, rows)],
                    outp_hbm.at[c, pl.ds(s * rows, rows)])


@functools.partial(
    pl.kernel,
    out_type=jax.ShapeDtypeStruct((NC, NP_OUT, C), jnp.float32),
    mesh=_mesh,
    scratch_types=[
        pltpu.VMEM((B,), jnp.int32),
        pltpu.VMEM((B,), jnp.int32),
        pltpu.VMEM((B, HC), jnp.float32),
        pltpu.VMEM((B * 4 + 16,), jnp.float32),
        pltpu.VMEM((B, C), jnp.float32),
        pltpu.VMEM_SHARED((NP_OUT, C), jnp.float32),
        pltpu.SemaphoreType.DMA,
    ],
    compiler_params=pltpu.CompilerParams(needs_layout_passes=False),
)
def _phase3(*refs):
    _p3_body(*refs)


def _fin_body(p_ref, x_ref, b_ref, gw_ref, gb_ref, gms_ref, o_ref):
    p = p_ref[0, :N, :] + p_ref[1, :N, :]
    out0 = p * (1.0 / H) + b_ref[...]
    mean = jnp.mean(out0, axis=0, keepdims=True)
    outc = out0 - gms_ref[...] * mean
    var = jnp.mean(outc * outc, axis=0, keepdims=True)
    y = outc * lax.rsqrt(var + 1e-5) * gw_ref[...] + gb_ref[...]
    y = jnp.where(y > 0, y, jnp.exp(y) - 1.0)
    o_ref[...] = y + x_ref[...]


def _final(outp, x, bias, gn_weight, gn_bias, gn_mean_scale):
    return pl.pallas_call(
        _fin_body,
        out_shape=jax.ShapeDtypeStruct((N, C), jnp.float32),
    )(outp, x, bias.reshape(1, C), gn_weight.reshape(1, C),
      gn_bias.reshape(1, C), gn_mean_scale.reshape(1, C))


def kernel(x, edge_index, W_l, b_l, W_r, b_r, att, bias, gn_weight,
           gn_bias, gn_mean_scale):
    ei = edge_index.astype(jnp.int32)
    pad = E_PAD - E
    src = jnp.concatenate([ei[0], jnp.zeros((pad,), jnp.int32)])
    dst = jnp.concatenate([ei[1], jnp.full((pad,), N, jnp.int32)])

    xl, xr = _matmuls(x, W_l, W_r, b_l, b_r)
    ex, den = _phase1(xl, xr, src, dst, att.reshape(HC))
    rden = _mid(den)
    al = _phase2(ex, dst, rden.reshape(ND))
    outp = _phase3(xl, src, dst, al)
    return _final(outp, x, bias, gn_weight, gn_bias, gn_mean_scale)


# speedup vs baseline: 19.2147x; 1.7233x over previous
import functools

import jax
import jax.numpy as jnp
from jax import lax
from jax.experimental import pallas as pl
from jax.experimental.pallas import tpu as pltpu
from jax.experimental.pallas import tpu_sc as plsc

N = 10000
E = 320000
C = 128
H = 4
HC = H * C

NC = 2
NS = 16
NW = NC * NS
EPT = 10048
B = 32
NBLK = EPT // B
E_PAD = NW * EPT
ND = H * EPT
NP_OUT = 10240

_mesh = plsc.VectorSubcoreMesh(
    core_axis_name="c", subcore_axis_name="s", num_cores=NC, num_subcores=NS)


def _mm_body(x_ref, wl_ref, wr_ref, bl_ref, br_ref, xl_ref, xr_ref):
    xv = x_ref[...]
    xl_ref[...] = jnp.dot(xv, wl_ref[...],
                          preferred_element_type=jnp.float32) + bl_ref[...]
    xr_ref[...] = jnp.dot(xv, wr_ref[...],
                          preferred_element_type=jnp.float32) + br_ref[...]


def _matmuls(x, W_l, W_r, b_l, b_r):
    blk = 1000
    grid = (N // blk,)
    return pl.pallas_call(
        _mm_body,
        grid=grid,
        in_specs=[
            pl.BlockSpec((blk, C), lambda i: (i, 0)),
            pl.BlockSpec((C, HC), lambda i: (0, 0)),
            pl.BlockSpec((C, HC), lambda i: (0, 0)),
            pl.BlockSpec((1, HC), lambda i: (0, 0)),
            pl.BlockSpec((1, HC), lambda i: (0, 0)),
        ],
        out_specs=[
            pl.BlockSpec((blk, HC), lambda i: (i, 0)),
            pl.BlockSpec((blk, HC), lambda i: (i, 0)),
        ],
        out_shape=[
            jax.ShapeDtypeStruct((N, HC), jnp.float32),
            jax.ShapeDtypeStruct((N, HC), jnp.float32),
        ],
    )(x, W_l, W_r, b_l.reshape(1, HC), b_r.reshape(1, HC))


def _p1_body(xl_hbm, xr_hbm, pk_hbm, att_hbm,
             ex_hbm, den_hbm,
             idxb, xlbuf, xrbuf, lscr, exblk, att_v, den_v,
             sem1, sem2, semi, seme):
    c = lax.axis_index("c")
    s = lax.axis_index("s")
    wid = s * NC + c

    pltpu.sync_copy(att_hbm, att_v)
    attv = [att_v[pl.ds(16 * j, 16)] for j in range(32)]
    zeros16 = jnp.zeros((16,), jnp.float32)

    @pl.loop(0, ND // 16)
    def _zero(i):
        den_v[pl.ds(i * 16, 16)] = zeros16

    gblk0 = wid * NBLK
    iota = lax.iota(jnp.int32, 16)
    onehot0 = (iota == 0).astype(jnp.float32)

    def start_gathers(bi, bg):
        pltpu.async_copy(xl_hbm.at[idxb.at[bi, 0]], xlbuf.at[bg], sem1)
        pltpu.async_copy(xr_hbm.at[idxb.at[bi, 1]], xrbuf.at[bg], sem2)

    def wait_gathers(bi, bg):
        pltpu.make_async_copy(xl_hbm.at[idxb.at[bi, 0]], xlbuf.at[bg], sem1).wait()
        pltpu.make_async_copy(xr_hbm.at[idxb.at[bi, 1]], xrbuf.at[bg], sem2).wait()

    pltpu.sync_copy(pk_hbm.at[gblk0], idxb.at[0])
    start_gathers(0, 0)
    pltpu.async_copy(pk_hbm.at[gblk0 + 1], idxb.at[1], semi)

    @pl.loop(0, NBLK)
    def _blk(blk):
        par = blk & 1
        parn = 1 - par
        i_cur = lax.rem(blk, 3)
        i_next = lax.rem(blk + 1, 3)
        i_pref = lax.rem(blk + 2, 3)
        wait_gathers(i_cur, par)

        @pl.when(blk + 1 < NBLK)
        def _next():
            pltpu.make_async_copy(
                pk_hbm.at[gblk0 + blk + 1], idxb.at[i_next], semi).wait()
            start_gathers(i_next, parn)

        @pl.when(blk + 2 < NBLK)
        def _pref():
            pltpu.async_copy(pk_hbm.at[gblk0 + blk + 2], idxb.at[i_pref], semi)

        @pl.loop(0, B)
        def _edge(e):
            for h in range(H):
                acc = zeros16
                for j8 in range(8):
                    j = h * 8 + j8
                    sv = (xlbuf[par, e, pl.ds(16 * j, 16)]
                          + xrbuf[par, e, pl.ds(16 * j, 16)])
                    lv = jnp.maximum(sv, 0.2 * sv)
                    acc = acc + lv * attv[j]
                lscr[pl.ds(e * 64 + h * 16, 16)] = acc

        @pl.when(blk >= 2)
        def _draine():
            pltpu.make_async_copy(
                exblk.at[pl.ds(0, B * 4)], ex_hbm.at[pl.ds(0, B * 4)], seme).wait()

        for g in range(2):
            dstv = idxb[i_cur, 1, pl.ds(g * 16, 16)]
            for h in range(H):
                tot = zeros16
                for j in range(16):
                    tot = tot + plsc.load_gather(
                        lscr, [iota * 64 + (g * 1024 + h * 16 + j)])
                exv = jnp.exp(tot)
                plsc.store_scatter(
                    exblk, [par * B * 4 + iota * 4 + (g * 64 + h)], exv)
                for j in range(16):
                    idx = dstv[j] * 4 + h
                    plsc.addupdate(den_v.at[pl.ds(idx, 16)], exv[j] * onehot0)

        base = (gblk0 + blk) * B
        pltpu.async_copy(exblk.at[pl.ds(par * B * 4, B * 4)],
                         ex_hbm.at[pl.ds(base * 4, B * 4)], seme)

    pltpu.make_async_copy(
        exblk.at[pl.ds(0, B * 4)], ex_hbm.at[pl.ds(0, B * 4)], seme).wait()
    pltpu.make_async_copy(
        exblk.at[pl.ds(0, B * 4)], ex_hbm.at[pl.ds(0, B * 4)], seme).wait()
    pltpu.sync_copy(den_v, den_hbm.at[wid])


@functools.partial(
    pl.kernel,
    out_type=(
        jax.ShapeDtypeStruct((E_PAD * 4,), jnp.float32),
        jax.ShapeDtypeStruct((NW, ND), jnp.float32),
    ),
    mesh=_mesh,
    scratch_types=[
        pltpu.VMEM((3, 2, B), jnp.int32),
        pltpu.VMEM((2, B, HC), jnp.float32),
        pltpu.VMEM((2, B, HC), jnp.float32),
        pltpu.VMEM((B * 64,), jnp.float32),
        pltpu.VMEM((2 * B * 4,), jnp.float32),
        pltpu.VMEM((HC,), jnp.float32),
        pltpu.VMEM((ND,), jnp.float32),
        pltpu.SemaphoreType.DMA,
        pltpu.SemaphoreType.DMA,
        pltpu.SemaphoreType.DMA,
        pltpu.SemaphoreType.DMA,
    ],
    compiler_params=pltpu.CompilerParams(needs_layout_passes=False),
)
def _phase1(*refs):
    _p1_body(*refs)


def _mid_body(den_ref, rden_ref):
    d = jnp.sum(den_ref[...], axis=0) + 1e-16
    rden_ref[...] = 1.0 / d


def _mid(den):
    return pl.pallas_call(
        _mid_body,
        out_shape=jax.ShapeDtypeStruct((ND // 128, 128), jnp.float32),
    )(den.reshape(NW, ND // 128, 128))


def _p2_body(ex_hbm, dst_hbm, rden_hbm, al_hbm, didx, exblk, rden_v):
    c = lax.axis_index("c")
    s = lax.axis_index("s")
    wid = s * NC + c
    base = wid * EPT

    pltpu.sync_copy(rden_hbm, rden_v)
    pltpu.sync_copy(dst_hbm.at[pl.ds(base, EPT)], didx)
    pltpu.sync_copy(ex_hbm.at[pl.ds(base * 4, EPT * 4)], exblk)
    iota = lax.iota(jnp.int32, 16)

    @pl.loop(0, EPT // 16)
    def _grp(g):
        dstv = didx[pl.ds(g * 16, 16)]
        dst4 = dstv * 4
        for h in range(H):
            exv = plsc.load_gather(exblk, [iota * 4 + (g * 64 + h)])
            rd = plsc.load_gather(rden_v, [dst4 + h])
            plsc.store_scatter(exblk, [iota * 4 + (g * 64 + h)], exv * rd)

    pltpu.sync_copy(exblk, al_hbm.at[pl.ds(base * 4, EPT * 4)])


@functools.partial(
    pl.kernel,
    out_type=jax.ShapeDtypeStruct((E_PAD * 4,), jnp.float32),
    mesh=_mesh,
    scratch_types=[
        pltpu.VMEM((EPT,), jnp.int32),
        pltpu.VMEM((EPT * 4,), jnp.float32),
        pltpu.VMEM((ND,), jnp.float32),
    ],
    compiler_params=pltpu.CompilerParams(needs_layout_passes=False),
)
def _phase2(*refs):
    _p2_body(*refs)


def _p3_body(xl_hbm, pk_hbm, al_hbm,
             outp_hbm,
             idxb, xlbuf, albuf, vbuf, out_sh, sem1, sema, semi):
    c = lax.axis_index("c")
    s = lax.axis_index("s")
    wid = s * NC + c

    zeros16 = jnp.zeros((16,), jnp.float32)

    @pl.loop(0, B)
    def _zv(e):
        for j8 in range(8):
            vbuf[e, pl.ds(16 * j8, 16)] = zeros16

    @pl.loop(0, NP_OUT // (NS * B))
    def _zo(t):
        pltpu.sync_copy(vbuf, out_sh.at[pl.ds(s * (NP_OUT // NS) + t * B, B)])

    plsc.subcore_barrier()

    gblk0 = wid * NBLK

    def start_fetch(bi, bg, gblk):
        pltpu.async_copy(xl_hbm.at[idxb.at[bi, 0]], xlbuf.at[bg], sem1)
        pltpu.async_copy(al_hbm.at[pl.ds(gblk * (B * 4), B * 4)],
                         albuf.at[bg, pl.ds(0, B * 4)], sema)

    def wait_fetch(bi, bg):
        pltpu.make_async_copy(xl_hbm.at[idxb.at[bi, 0]], xlbuf.at[bg], sem1).wait()
        pltpu.make_async_copy(al_hbm.at[pl.ds(0, B * 4)],
                              albuf.at[bg, pl.ds(0, B * 4)], sema).wait()

    pltpu.sync_copy(pk_hbm.at[gblk0], idxb.at[0])
    start_fetch(0, 0, gblk0)
    pltpu.async_copy(pk_hbm.at[gblk0 + 1], idxb.at[1], semi)

    @pl.loop(0, NBLK)
    def _blk(blk):
        par = blk & 1
        parn = 1 - par
        i_cur = lax.rem(blk, 3)
        i_next = lax.rem(blk + 1, 3)
        i_pref = lax.rem(blk + 2, 3)
        wait_fetch(i_cur, par)

        @pl.when(blk + 1 < NBLK)
        def _next():
            pltpu.make_async_copy(
                pk_hbm.at[gblk0 + blk + 1], idxb.at[i_next], semi).wait()
            start_fetch(i_next, parn, gblk0 + blk + 1)

        @pl.when(blk + 2 < NBLK)
        def _pref():
            pltpu.async_copy(pk_hbm.at[gblk0 + blk + 2], idxb.at[i_pref], semi)

        @pl.loop(0, B)
        def _edge(e):
            av = albuf[par, pl.ds(e * 4, 16)]
            a0 = av[0]
            a1 = av[1]
            a2 = av[2]
            a3 = av[3]
            for c8 in range(8):
                v = (a0 * xlbuf[par, e, pl.ds(c8 * 16, 16)]
                     + a1 * xlbuf[par, e, pl.ds(128 + c8 * 16, 16)]
                     + a2 * xlbuf[par, e, pl.ds(256 + c8 * 16, 16)]
                     + a3 * xlbuf[par, e, pl.ds(384 + c8 * 16, 16)])
                vbuf[e, pl.ds(c8 * 16, 16)] = v

        pltpu.sync_copy(vbuf, out_sh.at[idxb.at[i_cur, 1]], add=True)

    plsc.subcore_barrier()
    rows = NP_OUT // NS
    pltpu.sync_copy(out_sh.at[pl.ds(s * rows, rows)],
                    outp_hbm.at[c, pl.ds(s * rows, rows)])


@functools.partial(
    pl.kernel,
    out_type=jax.ShapeDtypeStruct((NC, NP_OUT, C), jnp.float32),
    mesh=_mesh,
    scratch_types=[
        pltpu.VMEM((3, 2, B), jnp.int32),
        pltpu.VMEM((2, B, HC), jnp.float32),
        pltpu.VMEM((2, B * 4 + 16), jnp.float32),
        pltpu.VMEM((B, C), jnp.float32),
        pltpu.VMEM_SHARED((NP_OUT, C), jnp.float32),
        pltpu.SemaphoreType.DMA,
        pltpu.SemaphoreType.DMA,
        pltpu.SemaphoreType.DMA,
    ],
    compiler_params=pltpu.CompilerParams(needs_layout_passes=False),
)
def _phase3(*refs):
    _p3_body(*refs)


def _fin_body(p_ref, x_ref, b_ref, gw_ref, gb_ref, gms_ref, o_ref):
    p = p_ref[0, :N, :] + p_ref[1, :N, :]
    out0 = p * (1.0 / H) + b_ref[...]
    mean = jnp.mean(out0, axis=0, keepdims=True)
    outc = out0 - gms_ref[...] * mean
    var = jnp.mean(outc * outc, axis=0, keepdims=True)
    y = outc * lax.rsqrt(var + 1e-5) * gw_ref[...] + gb_ref[...]
    y = jnp.where(y > 0, y, jnp.exp(y) - 1.0)
    o_ref[...] = y + x_ref[...]


def _final(outp, x, bias, gn_weight, gn_bias, gn_mean_scale):
    return pl.pallas_call(
        _fin_body,
        out_shape=jax.ShapeDtypeStruct((N, C), jnp.float32),
    )(outp, x, bias.reshape(1, C), gn_weight.reshape(1, C),
      gn_bias.reshape(1, C), gn_mean_scale.reshape(1, C))


def kernel(x, edge_index, W_l, b_l, W_r, b_r, att, bias, gn_weight,
           gn_bias, gn_mean_scale):
    ei = edge_index.astype(jnp.int32)
    pad = E_PAD - E
    src = jnp.concatenate([ei[0], jnp.zeros((pad,), jnp.int32)])
    dst = jnp.concatenate([ei[1], jnp.full((pad,), N, jnp.int32)])
    pk = jnp.stack([src.reshape(-1, B), dst.reshape(-1, B)], axis=1)

    xl, xr = _matmuls(x, W_l, W_r, b_l, b_r)
    ex, den = _phase1(xl, xr, pk, att.reshape(HC))
    rden = _mid(den)
    al = _phase2(ex, dst, rden.reshape(ND))
    outp = _phase3(xl, pk, al)
    return _final(outp, x, bias, gn_weight, gn_bias, gn_mean_scale)


# speedup vs baseline: 19.3469x; 1.0069x over previous
import functools

import jax
import jax.numpy as jnp
from jax import lax
from jax.experimental import pallas as pl
from jax.experimental.pallas import tpu as pltpu
from jax.experimental.pallas import tpu_sc as plsc

N = 10000
E = 320000
C = 128
H = 4
HC = H * C

NC = 2
NS = 16
NW = NC * NS
EPT = 10048
B = 32
NBLK = EPT // B
E_PAD = NW * EPT
ND = H * EPT
NP_OUT = 10240

_mesh = plsc.VectorSubcoreMesh(
    core_axis_name="c", subcore_axis_name="s", num_cores=NC, num_subcores=NS)


def _mm_body(x_ref, wl_ref, wr_ref, bl_ref, br_ref, xl_ref, xr_ref):
    xv = x_ref[...]
    xl_ref[...] = jnp.dot(xv, wl_ref[...],
                          preferred_element_type=jnp.float32) + bl_ref[...]
    xr_ref[...] = jnp.dot(xv, wr_ref[...],
                          preferred_element_type=jnp.float32) + br_ref[...]


def _matmuls(x, W_l, W_r, b_l, b_r):
    blk = 1000
    grid = (N // blk,)
    return pl.pallas_call(
        _mm_body,
        grid=grid,
        in_specs=[
            pl.BlockSpec((blk, C), lambda i: (i, 0)),
            pl.BlockSpec((C, HC), lambda i: (0, 0)),
            pl.BlockSpec((C, HC), lambda i: (0, 0)),
            pl.BlockSpec((1, HC), lambda i: (0, 0)),
            pl.BlockSpec((1, HC), lambda i: (0, 0)),
        ],
        out_specs=[
            pl.BlockSpec((blk, HC), lambda i: (i, 0)),
            pl.BlockSpec((blk, HC), lambda i: (i, 0)),
        ],
        out_shape=[
            jax.ShapeDtypeStruct((N, HC), jnp.float32),
            jax.ShapeDtypeStruct((N, HC), jnp.float32),
        ],
    )(x, W_l, W_r, b_l.reshape(1, HC), b_r.reshape(1, HC))


def _p1_body(xl_hbm, xr_hbm, pk_hbm, att_hbm,
             ex_hbm, den_hbm,
             idxb, xlbuf, xrbuf, lscr, exblk, att_v, den_v,
             sem1, sem2, semi, seme):
    c = lax.axis_index("c")
    s = lax.axis_index("s")
    wid = s * NC + c

    pltpu.sync_copy(att_hbm, att_v)
    attv = [att_v[pl.ds(16 * j, 16)] for j in range(32)]
    zeros16 = jnp.zeros((16,), jnp.float32)

    @pl.loop(0, ND // 16)
    def _zero(i):
        den_v[pl.ds(i * 16, 16)] = zeros16

    gblk0 = wid * NBLK
    iota = lax.iota(jnp.int32, 16)
    onehot0 = (iota == 0).astype(jnp.float32)

    def start_gathers(bi, bg):
        pltpu.async_copy(xl_hbm.at[idxb.at[bi, 0]], xlbuf.at[bg], sem1)
        pltpu.async_copy(xr_hbm.at[idxb.at[bi, 1]], xrbuf.at[bg], sem2)

    def wait_gathers(bi, bg):
        pltpu.make_async_copy(xl_hbm.at[idxb.at[bi, 0]], xlbuf.at[bg], sem1).wait()
        pltpu.make_async_copy(xr_hbm.at[idxb.at[bi, 1]], xrbuf.at[bg], sem2).wait()

    pltpu.sync_copy(pk_hbm.at[gblk0], idxb.at[0])
    start_gathers(0, 0)
    pltpu.async_copy(pk_hbm.at[gblk0 + 1], idxb.at[1], semi)

    @pl.loop(0, NBLK)
    def _blk(blk):
        par = blk & 1
        parn = 1 - par
        i_cur = lax.rem(blk, 3)
        i_next = lax.rem(blk + 1, 3)
        i_pref = lax.rem(blk + 2, 3)
        wait_gathers(i_cur, par)

        @pl.when(blk + 1 < NBLK)
        def _next():
            pltpu.make_async_copy(
                pk_hbm.at[gblk0 + blk + 1], idxb.at[i_next], semi).wait()
            start_gathers(i_next, parn)

        @pl.when(blk + 2 < NBLK)
        def _pref():
            pltpu.async_copy(pk_hbm.at[gblk0 + blk + 2], idxb.at[i_pref], semi)

        @pl.loop(0, B, unroll=4)
        def _edge(e):
            for h in range(H):
                acc = zeros16
                for j8 in range(8):
                    j = h * 8 + j8
                    sv = (xlbuf[par, e, pl.ds(16 * j, 16)]
                          + xrbuf[par, e, pl.ds(16 * j, 16)])
                    lv = jnp.maximum(sv, 0.2 * sv)
                    acc = acc + lv * attv[j]
                lscr[pl.ds(e * 64 + h * 16, 16)] = acc

        @pl.when(blk >= 2)
        def _draine():
            pltpu.make_async_copy(
                exblk.at[pl.ds(0, B * 4)], ex_hbm.at[pl.ds(0, B * 4)], seme).wait()

        for g in range(2):
            dstv = idxb[i_cur, 1, pl.ds(g * 16, 16)]
            for h in range(H):
                tot = zeros16
                for j in range(16):
                    tot = tot + plsc.load_gather(
                        lscr, [iota * 64 + (g * 1024 + h * 16 + j)])
                exv = jnp.exp(tot)
                plsc.store_scatter(
                    exblk, [par * B * 4 + iota * 4 + (g * 64 + h)], exv)
                for j in range(16):
                    idx = dstv[j] * 4 + h
                    plsc.addupdate(den_v.at[pl.ds(idx, 16)], exv[j] * onehot0)

        base = (gblk0 + blk) * B
        pltpu.async_copy(exblk.at[pl.ds(par * B * 4, B * 4)],
                         ex_hbm.at[pl.ds(base * 4, B * 4)], seme)

    pltpu.make_async_copy(
        exblk.at[pl.ds(0, B * 4)], ex_hbm.at[pl.ds(0, B * 4)], seme).wait()
    pltpu.make_async_copy(
        exblk.at[pl.ds(0, B * 4)], ex_hbm.at[pl.ds(0, B * 4)], seme).wait()
    pltpu.sync_copy(den_v, den_hbm.at[wid])


@functools.partial(
    pl.kernel,
    out_type=(
        jax.ShapeDtypeStruct((E_PAD * 4,), jnp.float32),
        jax.ShapeDtypeStruct((NW, ND), jnp.float32),
    ),
    mesh=_mesh,
    scratch_types=[
        pltpu.VMEM((3, 2, B), jnp.int32),
        pltpu.VMEM((2, B, HC), jnp.float32),
        pltpu.VMEM((2, B, HC), jnp.float32),
        pltpu.VMEM((B * 64,), jnp.float32),
        pltpu.VMEM((2 * B * 4,), jnp.float32),
        pltpu.VMEM((HC,), jnp.float32),
        pltpu.VMEM((ND,), jnp.float32),
        pltpu.SemaphoreType.DMA,
        pltpu.SemaphoreType.DMA,
        pltpu.SemaphoreType.DMA,
        pltpu.SemaphoreType.DMA,
    ],
    compiler_params=pltpu.CompilerParams(needs_layout_passes=False),
)
def _phase1(*refs):
    _p1_body(*refs)


def _mid_body(den_ref, rden_ref):
    d = jnp.sum(den_ref[...], axis=0) + 1e-16
    rden_ref[...] = 1.0 / d


def _mid(den):
    return pl.pallas_call(
        _mid_body,
        out_shape=jax.ShapeDtypeStruct((ND // 128, 128), jnp.float32),
    )(den.reshape(NW, ND // 128, 128))


def _p2_body(ex_hbm, dst_hbm, rden_hbm, al_hbm, didx, exblk, rden_v):
    c = lax.axis_index("c")
    s = lax.axis_index("s")
    wid = s * NC + c
    base = wid * EPT

    pltpu.sync_copy(rden_hbm, rden_v)
    pltpu.sync_copy(dst_hbm.at[pl.ds(base, EPT)], didx)
    pltpu.sync_copy(ex_hbm.at[pl.ds(base * 4, EPT * 4)], exblk)
    iota = lax.iota(jnp.int32, 16)

    @pl.loop(0, EPT // 16)
    def _grp(g):
        dstv = didx[pl.ds(g * 16, 16)]
        dst4 = dstv * 4
        for h in range(H):
            exv = plsc.load_gather(exblk, [iota * 4 + (g * 64 + h)])
            rd = plsc.load_gather(rden_v, [dst4 + h])
            plsc.store_scatter(exblk, [iota * 4 + (g * 64 + h)], exv * rd)

    pltpu.sync_copy(exblk, al_hbm.at[pl.ds(base * 4, EPT * 4)])


@functools.partial(
    pl.kernel,
    out_type=jax.ShapeDtypeStruct((E_PAD * 4,), jnp.float32),
    mesh=_mesh,
    scratch_types=[
        pltpu.VMEM((EPT,), jnp.int32),
        pltpu.VMEM((EPT * 4,), jnp.float32),
        pltpu.VMEM((ND,), jnp.float32),
    ],
    compiler_params=pltpu.CompilerParams(needs_layout_passes=False),
)
def _phase2(*refs):
    _p2_body(*refs)


def _p3_body(xl_hbm, pk_hbm, al_hbm,
             outp_hbm,
             idxb, xlbuf, albuf, vbuf, out_sh, sem1, sema, semi):
    c = lax.axis_index("c")
    s = lax.axis_index("s")
    wid = s * NC + c

    zeros16 = jnp.zeros((16,), jnp.float32)

    @pl.loop(0, B)
    def _zv(e):
        for j8 in range(8):
            vbuf[e, pl.ds(16 * j8, 16)] = zeros16

    @pl.loop(0, NP_OUT // (NS * B))
    def _zo(t):
        pltpu.sync_copy(vbuf, out_sh.at[pl.ds(s * (NP_OUT // NS) + t * B, B)])

    plsc.subcore_barrier()

    gblk0 = wid * NBLK

    def start_fetch(bi, bg, gblk):
        pltpu.async_copy(xl_hbm.at[idxb.at[bi, 0]], xlbuf.at[bg], sem1)
        pltpu.async_copy(al_hbm.at[pl.ds(gblk * (B * 4), B * 4)],
                         albuf.at[bg, pl.ds(0, B * 4)], sema)

    def wait_fetch(bi, bg):
        pltpu.make_async_copy(xl_hbm.at[idxb.at[bi, 0]], xlbuf.at[bg], sem1).wait()
        pltpu.make_async_copy(al_hbm.at[pl.ds(0, B * 4)],
                              albuf.at[bg, pl.ds(0, B * 4)], sema).wait()

    pltpu.sync_copy(pk_hbm.at[gblk0], idxb.at[0])
    start_fetch(0, 0, gblk0)
    pltpu.async_copy(pk_hbm.at[gblk0 + 1], idxb.at[1], semi)

    @pl.loop(0, NBLK)
    def _blk(blk):
        par = blk & 1
        parn = 1 - par
        i_cur = lax.rem(blk, 3)
        i_next = lax.rem(blk + 1, 3)
        i_pref = lax.rem(blk + 2, 3)
        wait_fetch(i_cur, par)

        @pl.when(blk + 1 < NBLK)
        def _next():
            pltpu.make_async_copy(
                pk_hbm.at[gblk0 + blk + 1], idxb.at[i_next], semi).wait()
            start_fetch(i_next, parn, gblk0 + blk + 1)

        @pl.when(blk + 2 < NBLK)
        def _pref():
            pltpu.async_copy(pk_hbm.at[gblk0 + blk + 2], idxb.at[i_pref], semi)

        @pl.loop(0, B, unroll=4)
        def _edge(e):
            av = albuf[par, pl.ds(e * 4, 16)]
            a0 = av[0]
            a1 = av[1]
            a2 = av[2]
            a3 = av[3]
            for c8 in range(8):
                v = (a0 * xlbuf[par, e, pl.ds(c8 * 16, 16)]
                     + a1 * xlbuf[par, e, pl.ds(128 + c8 * 16, 16)]
                     + a2 * xlbuf[par, e, pl.ds(256 + c8 * 16, 16)]
                     + a3 * xlbuf[par, e, pl.ds(384 + c8 * 16, 16)])
                vbuf[e, pl.ds(c8 * 16, 16)] = v

        pltpu.sync_copy(vbuf, out_sh.at[idxb.at[i_cur, 1]], add=True)

    plsc.subcore_barrier()
    rows = NP_OUT // NS
    pltpu.sync_copy(out_sh.at[pl.ds(s * rows, rows)],
                    outp_hbm.at[c, pl.ds(s * rows, rows)])


@functools.partial(
    pl.kernel,
    out_type=jax.ShapeDtypeStruct((NC, NP_OUT, C), jnp.float32),
    mesh=_mesh,
    scratch_types=[
        pltpu.VMEM((3, 2, B), jnp.int32),
        pltpu.VMEM((2, B, HC), jnp.float32),
        pltpu.VMEM((2, B * 4 + 16), jnp.float32),
        pltpu.VMEM((B, C), jnp.float32),
        pltpu.VMEM_SHARED((NP_OUT, C), jnp.float32),
        pltpu.SemaphoreType.DMA,
        pltpu.SemaphoreType.DMA,
        pltpu.SemaphoreType.DMA,
    ],
    compiler_params=pltpu.CompilerParams(needs_layout_passes=False),
)
def _phase3(*refs):
    _p3_body(*refs)


def _fin_body(p_ref, x_ref, b_ref, gw_ref, gb_ref, gms_ref, o_ref):
    p = p_ref[0, :N, :] + p_ref[1, :N, :]
    out0 = p * (1.0 / H) + b_ref[...]
    mean = jnp.mean(out0, axis=0, keepdims=True)
    outc = out0 - gms_ref[...] * mean
    var = jnp.mean(outc * outc, axis=0, keepdims=True)
    y = outc * lax.rsqrt(var + 1e-5) * gw_ref[...] + gb_ref[...]
    y = jnp.where(y > 0, y, jnp.exp(y) - 1.0)
    o_ref[...] = y + x_ref[...]


def _final(outp, x, bias, gn_weight, gn_bias, gn_mean_scale):
    return pl.pallas_call(
        _fin_body,
        out_shape=jax.ShapeDtypeStruct((N, C), jnp.float32),
    )(outp, x, bias.reshape(1, C), gn_weight.reshape(1, C),
      gn_bias.reshape(1, C), gn_mean_scale.reshape(1, C))


def kernel(x, edge_index, W_l, b_l, W_r, b_r, att, bias, gn_weight,
           gn_bias, gn_mean_scale):
    ei = edge_index.astype(jnp.int32)
    pad = E_PAD - E
    src = jnp.concatenate([ei[0], jnp.zeros((pad,), jnp.int32)])
    dst = jnp.concatenate([ei[1], jnp.full((pad,), N, jnp.int32)])
    pk = jnp.stack([src.reshape(-1, B), dst.reshape(-1, B)], axis=1)

    xl, xr = _matmuls(x, W_l, W_r, b_l, b_r)
    ex, den = _phase1(xl, xr, pk, att.reshape(HC))
    rden = _mid(den)
    al = _phase2(ex, dst, rden.reshape(ND))
    outp = _phase3(xl, pk, al)
    return _final(outp, x, bias, gn_weight, gn_bias, gn_mean_scale)
